# baseline (device time: 61272 ns/iter reference)
import jax
import jax.numpy as jnp
from jax import lax
from jax.experimental import pallas as pl
from jax.experimental.pallas import tpu as pltpu

NZ = 4


def kernel(partial, resid, gamma):
    _, m, d = partial.shape
    p2 = partial.reshape(m, d)
    g2 = gamma.reshape(1, d)
    rows = m // NZ
    half = rows // 2

    def body(p_ref, r_ref, g_ref, o_ref, ycomm, ysem_s, ysem_r, zsend, zrecv):
        my_x = lax.axis_index("x")
        my_y = lax.axis_index("y")
        my_z = lax.axis_index("z")
        py = (my_x, 1 - my_y, my_z)
        rz = lax.rem(my_z + 1, NZ)
        lz = lax.rem(my_z + NZ - 1, NZ)
        z2 = lax.rem(my_z + 2, NZ)
        R = (my_x, my_y, rz)
        L = (my_x, my_y, lz)

        barrier_sem = pltpu.get_barrier_semaphore()
        for nbr in (py, R, L):
            pl.semaphore_signal(
                barrier_sem, inc=1, device_id=nbr,
                device_id_type=pl.DeviceIdType.MESH,
            )
        pl.semaphore_wait(barrier_sem, 3)

        blk = pl.ds(my_z * rows, rows)

        rdma_y = pltpu.make_async_remote_copy(
            src_ref=p_ref.at[blk, :],
            dst_ref=ycomm,
            send_sem=ysem_s,
            recv_sem=ysem_r,
            device_id=py,
            device_id_type=pl.DeviceIdType.MESH,
        )
        rdma_y.start()
        rdma_y.wait()

        y = p_ref[blk, :] + ycomm[...] + r_ref[blk, :]
        rms = jnp.sqrt(jnp.mean(y * y, axis=-1, keepdims=True) + 1e-6)
        o_ref[blk, :] = y / rms * g_ref[...]

        s1r = pltpu.make_async_remote_copy(
            src_ref=o_ref.at[blk, :], dst_ref=o_ref.at[blk, :],
            send_sem=zsend.at[0], recv_sem=zrecv.at[0],
            device_id=R, device_id_type=pl.DeviceIdType.MESH,
        )
        s1l = pltpu.make_async_remote_copy(
            src_ref=o_ref.at[blk, :], dst_ref=o_ref.at[blk, :],
            send_sem=zsend.at[1], recv_sem=zrecv.at[1],
            device_id=L, device_id_type=pl.DeviceIdType.MESH,
        )
        s1r.start()
        s1l.start()

        lblk = pl.ds(lz * rows, rows)
        rblk = pl.ds(rz * rows, rows)
        pltpu.make_async_remote_copy(
            src_ref=o_ref.at[lblk, :], dst_ref=o_ref.at[lblk, :],
            send_sem=zsend.at[0], recv_sem=zrecv.at[0],
            device_id=L, device_id_type=pl.DeviceIdType.MESH,
        ).wait_recv()
        pltpu.make_async_remote_copy(
            src_ref=o_ref.at[rblk, :], dst_ref=o_ref.at[rblk, :],
            send_sem=zsend.at[1], recv_sem=zrecv.at[1],
            device_id=R, device_id_type=pl.DeviceIdType.MESH,
        ).wait_recv()

        fwdB = pl.ds(lz * rows + half, half)
        fwdA = pl.ds(rz * rows, half)
        s2r = pltpu.make_async_remote_copy(
            src_ref=o_ref.at[fwdB, :], dst_ref=o_ref.at[fwdB, :],
            send_sem=zsend.at[2], recv_sem=zrecv.at[2],
            device_id=R, device_id_type=pl.DeviceIdType.MESH,
        )
        s2l = pltpu.make_async_remote_copy(
            src_ref=o_ref.at[fwdA, :], dst_ref=o_ref.at[fwdA, :],
            send_sem=zsend.at[3], recv_sem=zrecv.at[3],
            device_id=L, device_id_type=pl.DeviceIdType.MESH,
        )
        s2r.start()
        s2l.start()

        z2B = pl.ds(z2 * rows + half, half)
        z2A = pl.ds(z2 * rows, half)
        pltpu.make_async_remote_copy(
            src_ref=o_ref.at[z2B, :], dst_ref=o_ref.at[z2B, :],
            send_sem=zsend.at[2], recv_sem=zrecv.at[2],
            device_id=L, device_id_type=pl.DeviceIdType.MESH,
        ).wait_recv()
        pltpu.make_async_remote_copy(
            src_ref=o_ref.at[z2A, :], dst_ref=o_ref.at[z2A, :],
            send_sem=zsend.at[3], recv_sem=zrecv.at[3],
            device_id=R, device_id_type=pl.DeviceIdType.MESH,
        ).wait_recv()

        s1r.wait_send()
        s1l.wait_send()
        s2r.wait_send()
        s2l.wait_send()

    return pl.pallas_call(
        body,
        out_shape=jax.ShapeDtypeStruct((m, d), jnp.float32),
        in_specs=[
            pl.BlockSpec(memory_space=pltpu.VMEM),
            pl.BlockSpec(memory_space=pltpu.VMEM),
            pl.BlockSpec(memory_space=pltpu.VMEM),
        ],
        out_specs=pl.BlockSpec(memory_space=pltpu.VMEM),
        scratch_shapes=[
            pltpu.VMEM((rows, d), jnp.float32),
            pltpu.SemaphoreType.DMA,
            pltpu.SemaphoreType.DMA,
            pltpu.SemaphoreType.DMA((4,)),
            pltpu.SemaphoreType.DMA((4,)),
        ],
        compiler_params=pltpu.CompilerParams(collective_id=0),
    )(p2, resid, g2)
